# baseline (speedup 1.0000x reference)
import functools

import jax
import jax.numpy as jnp
from jax import lax
from jax.experimental import pallas as pl
from jax.experimental.pallas import tpu as pltpu
from jax.experimental.pallas import tpu_sc as plsc

D = 16
CHUNK = 128


def _sc_geometry():
    try:
        info = plsc.get_sparse_core_info()
        return info.num_cores, info.num_subcores
    except Exception:
        return 2, 16


def _sc_gather(user_idx2d, item_idx2d, utab, itab, B):
    NC, NS = _sc_geometry()
    NW = NC * NS
    n_chunks = B // (NW * CHUNK)
    rows_per_w = n_chunks * CHUNK
    mesh = plsc.VectorSubcoreMesh(core_axis_name="c", subcore_axis_name="s")

    def body(uidx_hbm, iidx_hbm, utab_hbm, itab_hbm,
             out_u, out_ib, out_i,
             idx_u, idx_i, ru, rib, ri, sem):
        wid = lax.axis_index("s") * NC + lax.axis_index("c")
        cbase = wid * n_chunks
        pltpu.sync_copy(uidx_hbm.at[pl.ds(cbase, n_chunks)], idx_u)
        pltpu.sync_copy(iidx_hbm.at[pl.ds(cbase, n_chunks)], idx_i)
        copies = []
        for j in range(n_chunks):
            dst = pl.ds(j * CHUNK, CHUNK)
            copies.append(pltpu.async_copy(utab_hbm.at[idx_u.at[j]], ru.at[dst], sem))
            copies.append(pltpu.async_copy(utab_hbm.at[idx_i.at[j]], rib.at[dst], sem))
            copies.append(pltpu.async_copy(itab_hbm.at[idx_i.at[j]], ri.at[dst], sem))
        for c in copies:
            c.wait()
        base = wid * rows_per_w
        pltpu.sync_copy(ru, out_u.at[pl.ds(base, rows_per_w)])
        pltpu.sync_copy(rib, out_ib.at[pl.ds(base, rows_per_w)])
        pltpu.sync_copy(ri, out_i.at[pl.ds(base, rows_per_w)])

    out_sds = jax.ShapeDtypeStruct((B, D), jnp.float32)
    k = pl.kernel(
        body,
        out_type=(out_sds, out_sds, out_sds),
        mesh=mesh,
        scratch_types=[
            pltpu.VMEM((n_chunks, CHUNK), jnp.int32),
            pltpu.VMEM((n_chunks, CHUNK), jnp.int32),
            pltpu.VMEM((rows_per_w, D), jnp.float32),
            pltpu.VMEM((rows_per_w, D), jnp.float32),
            pltpu.VMEM((rows_per_w, D), jnp.float32),
            pltpu.SemaphoreType.DMA,
        ],
        compiler_params=pltpu.CompilerParams(use_tc_tiling_on_sc=False),
    )
    return k(user_idx2d, item_idx2d, utab, itab)


def _mlp_body(ru, rib, ri, w1t, b1, w2t, b2, w3t, b3, womf, womlp, bo, out):
    u = ru[...]
    x = jnp.concatenate([u, ri[...]], axis=1)
    hp = jax.lax.Precision.HIGHEST
    h = jnp.maximum(jnp.dot(x, w1t[...], precision=hp) + b1[...], 0.0)
    h = jnp.maximum(jnp.dot(h, w2t[...], precision=hp) + b2[...], 0.0)
    h = jnp.maximum(jnp.dot(h, w3t[...], precision=hp) + b3[...], 0.0)
    mf = u * rib[...]
    logit = (jnp.dot(mf, womf[...], precision=hp)
             + jnp.dot(h, womlp[...], precision=hp) + bo[...])
    out[...] = jax.nn.sigmoid(logit)


def _tc_mlp(ru, rib, ri, W1, b1, W2, b2, W3, b3, Wo, bo, B):
    blk = 2048
    grid = B // blk
    full = lambda shape: pl.BlockSpec(shape, lambda i: (0, 0))
    row = lambda: pl.BlockSpec((blk, D), lambda i: (i, 0))
    return pl.pallas_call(
        _mlp_body,
        grid=(grid,),
        in_specs=[
            row(), row(), row(),
            full((32, 32)), full((1, 32)),
            full((32, 16)), full((1, 16)),
            full((16, 8)), full((1, 8)),
            full((16, 1)), full((8, 1)), full((1, 1)),
        ],
        out_specs=pl.BlockSpec((blk, 1), lambda i: (i, 0)),
        out_shape=jax.ShapeDtypeStruct((B, 1), jnp.float32),
    )(ru, rib, ri,
      W1.T, b1.reshape(1, 32),
      W2.T, b2.reshape(1, 16),
      W3.T, b3.reshape(1, 8),
      Wo[:, :D].T, Wo[:, D:].T, bo.reshape(1, 1))


def kernel(user_input, item_input, mf_user_table, mf_item_table,
           W1, b1, W2, b2, W3, b3, Wo, bo):
    B = user_input.shape[0]
    uidx = user_input.reshape(B // CHUNK, CHUNK)
    iidx = item_input.reshape(B // CHUNK, CHUNK)
    ru, rib, ri = _sc_gather(uidx, iidx, mf_user_table, mf_item_table, B)
    return _tc_mlp(ru, rib, ri, W1, b1, W2, b2, W3, b3, Wo, bo, B)
